# Initial kernel scaffold; baseline (speedup 1.0000x reference)
import jax
import jax.numpy as jnp
from jax.experimental import pallas as pl


def kernel(x, params):
    raise NotImplementedError("write your pallas kernel here")


# speedup vs baseline: 1.2042x; 1.2042x over previous
import functools

import jax
import jax.numpy as jnp
from jax import lax
from jax.experimental import pallas as pl
from jax.experimental.pallas import tpu as pltpu
from jax.experimental.pallas import tpu_sc as plsc

_B = 2
_N = 1024
_K = 32
_NCLS = 15
_EPS = 1e-5
_NEG = 0.01
_M = _B * _K
_NT = _M * _N


def _pc(**kw):
    return pl.pallas_call(**kw)


def _knn_body(xall_ref, xt_ref, o_ref, *, R, k):
    b = pl.program_id(0)
    xall = xall_ref[0]
    xt = xt_ref[0]
    xx = jnp.sum(xall * xall, axis=0)
    xxn = jnp.sum(xt * xt, axis=1)
    g = lax.dot_general(xt, xall, (((1,), (0,)), ((), ())),
                        preferred_element_type=jnp.float32)
    pd = 2.0 * g - xxn[:, None] - xx[None, :]
    iota = lax.broadcasted_iota(jnp.int32, (R, _N), 1)
    work = pd
    cols = []
    for _ in range(k):
        mx = jnp.max(work, axis=1, keepdims=True)
        cand = jnp.where(work == mx, iota, _N)
        amin = jnp.min(cand, axis=1)
        cols.append(amin)
        work = jnp.where(iota == amin[:, None], -jnp.inf, work)
    out = jnp.stack(cols, axis=1)
    o_ref[0] = out + b * _N


def _knn_topk(x):
    Bb, Cp, Nn = x.shape
    R = 256
    xt = jnp.transpose(x, (0, 2, 1))
    return _pc(
        kernel=functools.partial(_knn_body, R=R, k=_K),
        grid=(Bb, Nn // R),
        in_specs=[
            pl.BlockSpec((1, Cp, Nn), lambda b, i: (b, 0, 0)),
            pl.BlockSpec((1, R, Cp), lambda b, i: (b, i, 0)),
        ],
        out_specs=pl.BlockSpec((1, R, _K), lambda b, i: (b, i, 0)),
        out_shape=jax.ShapeDtypeStruct((Bb, Nn, _K), jnp.int32),
    )(x, xt)


def _gather_sc(table, idx):
    V, D = table.shape
    Bk = idx.shape[0]
    info = plsc.get_sparse_core_info()
    NW = info.num_cores * info.num_subcores
    b_per_w = Bk // NW
    CH = 128
    nch = b_per_w // CH
    mesh = plsc.VectorSubcoreMesh(core_axis_name="c", subcore_axis_name="s")

    @functools.partial(
        pl.kernel, mesh=mesh,
        out_type=jax.ShapeDtypeStruct((Bk, D), jnp.float32),
        scratch_types=[
            pltpu.VMEM((CH,), jnp.int32),
            pltpu.VMEM((CH, D), jnp.float32),
            pltpu.SemaphoreType.DMA,
        ],
    )
    def k(table_hbm, idx_hbm, out_hbm, idx_v, rows_v, sem):
        wid = lax.axis_index("s") * info.num_cores + lax.axis_index("c")
        base = wid * b_per_w

        def step(c, carry):
            off = base + c * CH
            pltpu.sync_copy(idx_hbm.at[pl.ds(off, CH)], idx_v)
            pltpu.async_copy(table_hbm.at[idx_v], rows_v, sem).wait()
            pltpu.sync_copy(rows_v, out_hbm.at[pl.ds(off, CH)])
            return carry

        lax.fori_loop(0, nch, step, 0)

    return k(table, idx)


def _mm_body(x_ref, w_ref, b_ref, y_ref):
    y = lax.dot_general(w_ref[...], x_ref[...], (((1,), (0,)), ((), ())),
                        preferred_element_type=jnp.float32)
    y_ref[0] = y + b_ref[...][0][:, None]


def _matmul_m(X, W, b):
    Cin, LT = X.shape
    M = LT // _N
    Cout = W.shape[0]
    return _pc(
        kernel=_mm_body,
        grid=(M,),
        in_specs=[
            pl.BlockSpec((Cin, _N), lambda m: (0, m)),
            pl.BlockSpec((Cout, Cin), lambda m: (0, 0)),
            pl.BlockSpec((1, Cout), lambda m: (0, 0)),
        ],
        out_specs=pl.BlockSpec((1, Cout, _N), lambda m: (m, 0, 0)),
        out_shape=jax.ShapeDtypeStruct((M, Cout, _N), jnp.float32),
    )(X, W, b.reshape(1, Cout))


def _bn_stats(Y):
    mean = jnp.mean(Y, axis=(0, 2))
    var = jnp.var(Y, axis=(0, 2))
    return mean.reshape(1, -1), jnp.sqrt(var + _EPS).reshape(1, -1)


def _norm_act(y, mean, sc, g, be):
    yn = (y - mean) / sc * g + be
    return jnp.where(yn >= 0, yn, _NEG * yn)


def _seq_sum(p):
    acc = p[0]
    for i in range(1, p.shape[0]):
        acc = acc + p[i]
    return acc


def _attn_body(yf_ref, ya_ref, mf_ref, vf_ref, ma_ref, va_ref,
               g_ref, be_ref, o_ref, *, C):
    mf = mf_ref[0].reshape(1, C, 1)
    vf = vf_ref[0].reshape(1, C, 1)
    f = _norm_act(yf_ref[...], mf, vf, g_ref[0, :C].reshape(1, C, 1),
                  be_ref[0, :C].reshape(1, C, 1))
    ma = ma_ref[0].reshape(1, C, 1)
    va = va_ref[0].reshape(1, C, 1)
    a = _norm_act(ya_ref[...], ma, va, g_ref[0, C:].reshape(1, C, 1),
                  be_ref[0, C:].reshape(1, C, 1))
    amx = jnp.max(a, axis=1, keepdims=True)
    e = jnp.exp(a - amx)
    aw = e / jnp.sum(e, axis=1, keepdims=True)
    o_ref[0] = _seq_sum(aw * f)


def _attn_combine(Yf, Ya, mf, vf, ma, va, g, be, C, Lb):
    return _pc(
        kernel=functools.partial(_attn_body, C=C),
        grid=(_B, _N // Lb),
        in_specs=[
            pl.BlockSpec((_K, C, Lb), lambda b, j: (b, 0, j)),
            pl.BlockSpec((_K, C, Lb), lambda b, j: (b, 0, j)),
            pl.BlockSpec((1, C), lambda b, j: (0, 0)),
            pl.BlockSpec((1, C), lambda b, j: (0, 0)),
            pl.BlockSpec((1, C), lambda b, j: (0, 0)),
            pl.BlockSpec((1, C), lambda b, j: (0, 0)),
            pl.BlockSpec((1, 2 * C), lambda b, j: (0, 0)),
            pl.BlockSpec((1, 2 * C), lambda b, j: (0, 0)),
        ],
        out_specs=pl.BlockSpec((1, C, Lb), lambda b, j: (b, 0, j)),
        out_shape=jax.ShapeDtypeStruct((_B, C, _N), jnp.float32),
    )(Yf, Ya, mf, vf, ma, va, g.reshape(1, -1), be.reshape(1, -1))


def _max_body(y_ref, m_ref, v_ref, g_ref, be_ref, o_ref, *, C):
    m = m_ref[0].reshape(1, C, 1)
    v = v_ref[0].reshape(1, C, 1)
    yl = _norm_act(y_ref[...], m, v, g_ref[0].reshape(1, C, 1),
                   be_ref[0].reshape(1, C, 1))
    o_ref[0] = jnp.max(yl, axis=0)


def _max_combine(Y, m, v, g, be, C, Lb):
    return _pc(
        kernel=functools.partial(_max_body, C=C),
        grid=(_B, _N // Lb),
        in_specs=[
            pl.BlockSpec((_K, C, Lb), lambda b, j: (b, 0, j)),
            pl.BlockSpec((1, C), lambda b, j: (0, 0)),
            pl.BlockSpec((1, C), lambda b, j: (0, 0)),
            pl.BlockSpec((1, C), lambda b, j: (0, 0)),
            pl.BlockSpec((1, C), lambda b, j: (0, 0)),
        ],
        out_specs=pl.BlockSpec((1, C, Lb), lambda b, j: (b, 0, j)),
        out_shape=jax.ShapeDtypeStruct((_B, C, _N), jnp.float32),
    )(Y, m, v, g.reshape(1, -1), be.reshape(1, -1))


def _head_body(y9_ref, m9_ref, v9_ref, g9_ref, be9_ref,
               y10_ref, m10_ref, v10_ref, g10_ref, be10_ref,
               w11_ref, b11_ref, w12_ref, b12_ref,
               w13_ref, b13_ref, w14_ref, b14_ref, o_ref):
    col = lambda r: r[0].reshape(-1, 1)
    fc = _norm_act(y9_ref[...], col(m9_ref), col(v9_ref),
                   col(g9_ref), col(be9_ref))
    fn = _norm_act(y10_ref[...], col(m10_ref), col(v10_ref),
                   col(g10_ref), col(be10_ref))
    mm = lambda a, b: lax.dot_general(a, b, (((1,), (0,)), ((), ())),
                                      preferred_element_type=jnp.float32)
    A = mm(w14_ref[...], w13_ref[...])
    A = mm(A, w12_ref[...])
    A = mm(A, w11_ref[...])
    v = mm(w12_ref[...], b11_ref[...]) + b12_ref[...]
    v = mm(w13_ref[...], v) + b13_ref[...]
    v = mm(w14_ref[...], v) + b14_ref[...]
    Z = mm(A[:, :512], fc) + mm(A[:, 512:], fn) + v
    mx = jnp.max(Z, axis=0, keepdims=True)
    zl = Z - mx
    o_ref[...] = zl - jnp.log(jnp.sum(jnp.exp(zl), axis=0, keepdims=True))


def _head(Y9, m9, v9, g9, be9, Y10, m10, v10, g10, be10, p):
    LT = Y9.shape[1]
    args = [Y9, m9, v9, g9.reshape(1, -1), be9.reshape(1, -1),
            Y10, m10, v10, g10.reshape(1, -1), be10.reshape(1, -1),
            p["W11"], p["b11"].reshape(-1, 1), p["W12"], p["b12"].reshape(-1, 1),
            p["W13"], p["b13"].reshape(-1, 1), p["W14"], p["b14"].reshape(-1, 1)]
    return _pc(
        kernel=_head_body,
        out_shape=jax.ShapeDtypeStruct((_NCLS, LT), jnp.float32),
    )(*args)


def _build_X(gath, ctr, lo, hi):
    h = hi - lo
    nb = jnp.transpose(gath[..., lo:hi], (0, 3, 1, 2))
    ct = jnp.broadcast_to(ctr[:, lo:hi, :, None], (_B, h, _N, _K))
    pre = jnp.concatenate([nb, ct], axis=1)
    m = pre.reshape(-1, 2 * h, _N)
    return jnp.transpose(m, (1, 0, 2)).reshape(2 * h, _M * _N)


def _build_Xa(gath, ctr, h):
    nb = jnp.transpose(gath[..., :h], (0, 3, 1, 2))
    ct = jnp.broadcast_to(ctr[:, :h, :, None], (_B, h, _N, _K))
    a1 = nb.reshape(-1, h, _N)
    a2 = ct.reshape(-1, h, _N)
    pre = jnp.concatenate([a1, a1 - a2], axis=1)
    return jnp.transpose(pre, (1, 0, 2)).reshape(2 * h, _M * _N)


def _level(src_feats, idx_flat, p, i_f, i_a, i_n, c2):
    h = c2 // 2
    Dpad = max(128, c2)
    tbl = jnp.transpose(src_feats, (0, 2, 1)).reshape(_B * _N, c2)
    if c2 < Dpad:
        tbl = jnp.pad(tbl, ((0, 0), (0, Dpad - c2)))
    gath = _gather_sc(tbl, idx_flat)[:, :c2].reshape(_B, _N, _K, c2)

    Xc = _build_X(gath, src_feats, 0, h)
    Xn = _build_X(gath, src_feats, h, c2)
    Xa = _build_Xa(gath, src_feats, h)

    Cout = p[f"W{i_f}"].shape[0]
    Lb = 128 if Cout >= 256 else 256
    Yf = _matmul_m(Xc, p[f"W{i_f}"], p[f"b{i_f}"])
    mf, vf = _bn_stats(Yf)
    Ya = _matmul_m(Xa, p[f"W{i_a}"], p[f"b{i_a}"])
    ma, va = _bn_stats(Ya)
    gst = jnp.concatenate([p[f"g{i_f}"], p[f"g{i_a}"]])
    best = jnp.concatenate([p[f"be{i_f}"], p[f"be{i_a}"]])
    xc = _attn_combine(Yf, Ya, mf, vf, ma, va, gst, best, Cout, Lb)

    Yn = _matmul_m(Xn, p[f"W{i_n}"], p[f"b{i_n}"])
    mn, vn = _bn_stats(Yn)
    xn = _max_combine(Yn, mn, vn, p[f"g{i_n}"], p[f"be{i_n}"], Cout, Lb)
    return xc, xn


def kernel(x, params):
    p = params
    xpart = jnp.pad(x[:, 9:12, :], ((0, 0), (0, 5), (0, 0)))
    idx1 = _knn_topk(xpart).reshape(-1)
    xc1, xn1 = _level(x, idx1, p, 0, 1, 2, 24)

    idx2 = _knn_topk(xc1).reshape(-1)
    src2 = jnp.concatenate([xc1, xn1], axis=1)
    xc2, xn2 = _level(src2, idx2, p, 3, 4, 5, 128)

    idx3 = _knn_topk(xc2).reshape(-1)
    src3 = jnp.concatenate([xc2, xn2], axis=1)
    xc3, xn3 = _level(src3, idx3, p, 6, 7, 8, 256)

    Xc = jnp.transpose(jnp.concatenate([xc1, xc2, xc3], axis=1),
                       (1, 0, 2)).reshape(448, _B * _N)
    Xn = jnp.transpose(jnp.concatenate([xn1, xn2, xn3], axis=1),
                       (1, 0, 2)).reshape(448, _B * _N)
    Y9 = _matmul_m(Xc, p["W9"], p["b9"])
    m9, v9 = _bn_stats(Y9)
    Y10 = _matmul_m(Xn, p["W10"], p["b10"])
    m10, v10 = _bn_stats(Y10)
    Y9f = jnp.transpose(Y9, (1, 0, 2)).reshape(512, _B * _N)
    Y10f = jnp.transpose(Y10, (1, 0, 2)).reshape(512, _B * _N)
    out = _head(Y9f, m9, v9, p["g9"], p["be9"],
                Y10f, m10, v10, p["g10"], p["be10"], p)
    return jnp.transpose(out.reshape(_NCLS, _B, _N), (1, 2, 0))
